# baseline (speedup 1.0000x reference)
import jax
import jax.numpy as jnp
from jax import lax
from jax.experimental import pallas as pl
from jax.experimental.pallas import tpu as pltpu
from jax.experimental.pallas import tpu_sc as plsc

_NC = 2
_NS = 16
_NW = _NC * _NS

_TOTAL_ROWS = 2 * 4096
_ROWS_PER_W = _TOTAL_ROWS // _NW
_SRC_OFFSET = 1 * 4096


def _copy_body(x_hbm, out_hbm):
    wid = lax.axis_index("s") * _NC + lax.axis_index("c")
    base = wid * _ROWS_PER_W
    pltpu.sync_copy(
        x_hbm.at[pl.ds(_SRC_OFFSET + base, _ROWS_PER_W)],
        out_hbm.at[pl.ds(base, _ROWS_PER_W)],
    )


def kernel(x):
    x2 = x.reshape(4 * 4096, 4096)
    mesh = plsc.VectorSubcoreMesh(core_axis_name="c", subcore_axis_name="s")
    out = pl.kernel(
        _copy_body,
        mesh=mesh,
        out_type=jax.ShapeDtypeStruct((_TOTAL_ROWS, 4096), jnp.float32),
    )(x2)
    return out.reshape(2, 4096, 4096)
